# baseline (speedup 1.0000x reference)
import functools

import jax
import jax.numpy as jnp
from jax import lax
from jax.experimental import pallas as pl
from jax.experimental.pallas import tpu as pltpu
from jax.experimental.pallas import tpu_sc as plsc

_N = 4194304
_NW = 32
_PER_W = _N // _NW
_CHUNK = 8192
_NCHUNK = _PER_W // _CHUNK
_L = 16
_UNROLL = 8
_NBUF = 3


def _sc_body(inp_hbm, tgt_hbm, out_hbm,
             ib0, ib1, ib2, tb0, tb1, tb2, accv, s0, s1, s2):
    wid = lax.axis_index("s") * 2 + lax.axis_index("c")
    base = pl.multiple_of(wid * _PER_W, _PER_W)
    ibufs = (ib0, ib1, ib2)
    tbufs = (tb0, tb1, tb2)
    sems = (s0, s1, s2)
    h_i = [None] * _NBUF
    h_t = [None] * _NBUF
    for c in range(_NBUF - 1):
        off = base + c * _CHUNK
        h_i[c] = pltpu.async_copy(
            inp_hbm.at[pl.ds(off, _CHUNK)], ibufs[c], sems[c])
        h_t[c] = pltpu.async_copy(
            tgt_hbm.at[pl.ds(off, _CHUNK)], tbufs[c], sems[c])
    acc = jnp.zeros((_L,), jnp.float32)
    for c in range(_NCHUNK):
        cur = c % _NBUF
        nxt = (c + _NBUF - 1) % _NBUF
        if c + _NBUF - 1 < _NCHUNK:
            off = base + (c + _NBUF - 1) * _CHUNK
            h_i[nxt] = pltpu.async_copy(
                inp_hbm.at[pl.ds(off, _CHUNK)], ibufs[nxt], sems[nxt])
            h_t[nxt] = pltpu.async_copy(
                tgt_hbm.at[pl.ds(off, _CHUNK)], tbufs[nxt], sems[nxt])
        h_i[cur].wait()
        h_t[cur].wait()
        ibuf = ibufs[cur]
        tbuf = tbufs[cur]

        def _vec_body(i, a, ibuf=ibuf, tbuf=tbuf):
            j = i * (_UNROLL * _L)
            for u in range(_UNROLL):
                x = ibuf[pl.ds(j + u * _L, _L)]
                t = tbuf[pl.ds(j + u * _L, _L)]
                d = t - x
                a = a + d * d
            return a

        acc = lax.fori_loop(0, _CHUNK // (_UNROLL * _L), _vec_body, acc)
    accv[...] = acc * (1.0 / _N)
    pltpu.sync_copy(accv, out_hbm.at[wid])


_sc_mse = functools.partial(
    pl.kernel,
    mesh=plsc.VectorSubcoreMesh(core_axis_name="c", subcore_axis_name="s"),
    out_type=jax.ShapeDtypeStruct((_NW, _L), jnp.float32),
    scratch_types=(
        [pltpu.VMEM((_CHUNK,), jnp.float32)] * 6
        + [pltpu.VMEM((_L,), jnp.float32)]
        + [pltpu.SemaphoreType.DMA] * 3
    ),
)(_sc_body)


def _final_body(p_ref, o_ref):
    o_ref[...] = jnp.sum(p_ref[...]).reshape(1, 1)


def kernel(input, target):
    parts = _sc_mse(input, target)
    out = pl.pallas_call(
        _final_body,
        out_shape=jax.ShapeDtypeStruct((1, 1), jnp.float32),
    )(parts)
    return out[0, 0]


# speedup vs baseline: 1.0140x; 1.0140x over previous
import functools

import jax
import jax.numpy as jnp
from jax import lax
from jax.experimental import pallas as pl
from jax.experimental.pallas import tpu as pltpu
from jax.experimental.pallas import tpu_sc as plsc

_N = 4194304
_NW = 32
_PER_W = _N // _NW
_CHUNK = 8192
_NCHUNK = _PER_W // _CHUNK
_L = 16
_UNROLL = 8
_NBUF = 4


def _sc_body(inp_hbm, tgt_hbm, out_hbm,
             ib0, ib1, ib2, ib3, tb0, tb1, tb2, tb3, accv, s0, s1, s2, s3):
    wid = lax.axis_index("s") * 2 + lax.axis_index("c")
    base = pl.multiple_of(wid * _PER_W, _PER_W)
    ibufs = (ib0, ib1, ib2, ib3)
    tbufs = (tb0, tb1, tb2, tb3)
    sems = (s0, s1, s2, s3)
    h_i = [None] * _NBUF
    h_t = [None] * _NBUF
    for c in range(_NBUF - 1):
        off = base + c * _CHUNK
        h_i[c] = pltpu.async_copy(
            inp_hbm.at[pl.ds(off, _CHUNK)], ibufs[c], sems[c])
        h_t[c] = pltpu.async_copy(
            tgt_hbm.at[pl.ds(off, _CHUNK)], tbufs[c], sems[c])
    acc = jnp.zeros((_L,), jnp.float32)
    for c in range(_NCHUNK):
        cur = c % _NBUF
        nxt = (c + _NBUF - 1) % _NBUF
        if c + _NBUF - 1 < _NCHUNK:
            off = base + (c + _NBUF - 1) * _CHUNK
            h_i[nxt] = pltpu.async_copy(
                inp_hbm.at[pl.ds(off, _CHUNK)], ibufs[nxt], sems[nxt])
            h_t[nxt] = pltpu.async_copy(
                tgt_hbm.at[pl.ds(off, _CHUNK)], tbufs[nxt], sems[nxt])
        h_i[cur].wait()
        h_t[cur].wait()
        ibuf = ibufs[cur]
        tbuf = tbufs[cur]

        def _vec_body(i, a, ibuf=ibuf, tbuf=tbuf):
            j = i * (_UNROLL * _L)
            for u in range(_UNROLL):
                x = ibuf[pl.ds(j + u * _L, _L)]
                t = tbuf[pl.ds(j + u * _L, _L)]
                d = t - x
                a = a + d * d
            return a

        acc = lax.fori_loop(0, _CHUNK // (_UNROLL * _L), _vec_body, acc)
    accv[...] = acc * (1.0 / _N)
    pltpu.sync_copy(accv, out_hbm.at[wid])


_sc_mse = functools.partial(
    pl.kernel,
    mesh=plsc.VectorSubcoreMesh(core_axis_name="c", subcore_axis_name="s"),
    out_type=jax.ShapeDtypeStruct((_NW, _L), jnp.float32),
    scratch_types=(
        [pltpu.VMEM((_CHUNK,), jnp.float32)] * 8
        + [pltpu.VMEM((_L,), jnp.float32)]
        + [pltpu.SemaphoreType.DMA] * 4
    ),
)(_sc_body)


def _final_body(p_ref, o_ref):
    o_ref[...] = jnp.sum(p_ref[...]).reshape(1, 1)


def kernel(input, target):
    parts = _sc_mse(input, target)
    out = pl.pallas_call(
        _final_body,
        out_shape=jax.ShapeDtypeStruct((1, 1), jnp.float32),
    )(parts)
    return out[0, 0]
